# baseline (speedup 1.0000x reference)
import functools

import jax
import jax.numpy as jnp
from jax import lax
from jax.experimental import pallas as pl
from jax.experimental.pallas import tpu as pltpu
from jax.experimental.pallas import tpu_sc as plsc

N = 10000
E = 320000
D_IN = 128
H1 = 128
H2 = 128
D_OUT = 64

NSC = 2
NTILE = 16
RPT = 632
NR = NTILE * RPT
DUMMY = N
K = 128
CPT = 80
CROWS = 2560
CPA = 160
PH = 80
EPAD = CROWS * K

RB = 632
RB3 = 1000

_MESH = plsc.VectorSubcoreMesh(core_axis_name="c", subcore_axis_name="s")


@functools.partial(
    pl.kernel,
    out_type=(
        jax.ShapeDtypeStruct((NSC, NR, 16), jnp.float32),
        jax.ShapeDtypeStruct((NSC, NR, 16), jnp.float32),
    ),
    mesh=_MESH,
    scratch_types=[
        pltpu.VMEM((CPT, K), jnp.int32),
        pltpu.VMEM((CPT, K), jnp.int32),
        pltpu.VMEM((K, 16), jnp.float32),
        pltpu.VMEM_SHARED((NR, 16), jnp.float32),
        pltpu.VMEM_SHARED((NR, 16), jnp.float32),
    ],
    compiler_params=pltpu.CompilerParams(use_tc_tiling_on_sc=False),
)
def _deg_kernel(srcp2, dstp2, z16, dego, degi, sidx, didx, ones, acc_o, acc_i):
    c = lax.axis_index("c")
    s = lax.axis_index("s")
    tid = c * NTILE + s

    pltpu.sync_copy(srcp2.at[pl.ds(tid * CPT, CPT)], sidx)
    pltpu.sync_copy(dstp2.at[pl.ds(tid * CPT, CPT)], didx)
    pltpu.sync_copy(z16, acc_o.at[pl.ds(s * RPT, RPT)])
    pltpu.sync_copy(z16, acc_i.at[pl.ds(s * RPT, RPT)])

    for r in range(K):
        ones[r] = jnp.ones((16,), jnp.float32)

    plsc.subcore_barrier()

    @pl.loop(0, CPT)
    def _(i):
        pltpu.sync_copy(ones, acc_o.at[sidx.at[i]], add=True)
        pltpu.sync_copy(ones, acc_i.at[didx.at[i]], add=True)

    plsc.subcore_barrier()
    off = s * RPT
    pltpu.sync_copy(acc_o.at[pl.ds(off, RPT)], dego.at[c, pl.ds(off, RPT)])
    pltpu.sync_copy(acc_i.at[pl.ds(off, RPT)], degi.at[c, pl.ds(off, RPT)])


def _make_agg(D):
    @functools.partial(
        pl.kernel,
        out_type=jax.ShapeDtypeStruct((NR, D), jnp.float32),
        mesh=_MESH,
        scratch_types=[
            pltpu.VMEM((PH, K), jnp.int32),
            pltpu.VMEM((K,), jnp.int32),
            pltpu.VMEM((K,), jnp.int32),
            pltpu.VMEM((K, D), jnp.float32),
            pltpu.VMEM((K, D), jnp.float32),
            pltpu.VMEM_SHARED((NR, D), jnp.float32),
            pltpu.SemaphoreType.DMA,
            pltpu.SemaphoreType.DMA,
            pltpu.SemaphoreType.DMA,
            pltpu.SemaphoreType.DMA,
        ],
        compiler_params=pltpu.CompilerParams(use_tc_tiling_on_sc=False),
    )
    def agg(srcp2, dstp2, zrows, y, out, sidx, d0b, d1b, r0, r1, acc,
            g0, g1, dm0, dm1):
        rows = [r0, r1]
        gsem = [g0, g1]
        dbuf = [d0b, d1b]
        dsem = [dm0, dm1]
        c = lax.axis_index("c")
        s = lax.axis_index("s")

        @pl.when(c == 0)
        def _():
            pltpu.sync_copy(zrows, acc.at[pl.ds(s * RPT, RPT)])
            plsc.subcore_barrier()

            for ph in range(CPA // PH):
                base = s * CPA + ph * PH
                pltpu.sync_copy(srcp2.at[pl.ds(base, PH)], sidx)
                pltpu.async_copy(dstp2.at[base], dbuf[0], dsem[0])
                pltpu.async_copy(y.at[sidx.at[0]], rows[0], gsem[0])
                pltpu.async_copy(y.at[sidx.at[1]], rows[1], gsem[1])

                @pl.loop(0, PH // 2)
                def _(j):
                    i0 = j * 2
                    for b in range(2):
                        i = i0 + b

                        @pl.when(i + 1 < PH)
                        def _():
                            pltpu.async_copy(dstp2.at[base + i + 1],
                                             dbuf[1 - b], dsem[1 - b])

                        pltpu.make_async_copy(y.at[sidx.at[i]], rows[b],
                                              gsem[b]).wait()
                        pltpu.make_async_copy(dstp2.at[base + i], dbuf[b],
                                              dsem[b]).wait()
                        pltpu.sync_copy(rows[b], acc.at[dbuf[b]], add=True)

                        @pl.when(i + 2 < PH)
                        def _():
                            pltpu.async_copy(y.at[sidx.at[i + 2]], rows[b],
                                             gsem[b])

            plsc.subcore_barrier()
            pltpu.sync_copy(acc.at[pl.ds(s * RPT, RPT)],
                            out.at[pl.ds(s * RPT, RPT)])

    return agg


_agg128 = _make_agg(128)
_agg64 = _make_agg(64)


def _norm(deg):
    return jnp.where(deg > 0, lax.rsqrt(jnp.maximum(deg, 1.0)), 0.0)


def _stage0_body(x_ref, dego_ref, w_ref, o_ref):
    deg = dego_ref[0, :, 0:1] + dego_ref[1, :, 0:1]
    h = x_ref[...] * _norm(deg)
    o_ref[...] = jnp.dot(h, w_ref[...], preferred_element_type=jnp.float32,
                       precision=lax.Precision.HIGHEST)


def _stage0(x, dego, w):
    return pl.pallas_call(
        _stage0_body,
        grid=(NR // RB,),
        in_specs=[
            pl.BlockSpec((RB, D_IN), lambda i: (i, 0)),
            pl.BlockSpec((NSC, RB, 16), lambda i: (0, i, 0)),
            pl.BlockSpec((D_IN, H1), lambda i: (0, 0)),
        ],
        out_specs=pl.BlockSpec((RB, H1), lambda i: (i, 0)),
        out_shape=jax.ShapeDtypeStruct((NR, H1), jnp.float32),
    )(x, dego, w)


def _stage_mid_body(p_ref, degi_ref, dego_ref, b_ref, w_ref, o_ref):
    di = degi_ref[0, :, 0:1] + degi_ref[1, :, 0:1]
    do = dego_ref[0, :, 0:1] + dego_ref[1, :, 0:1]
    agg = p_ref[...] * _norm(di) + b_ref[...]
    h = jnp.where(agg >= 0, agg, 0.1 * agg) * _norm(do)
    o_ref[...] = jnp.dot(h, w_ref[...], preferred_element_type=jnp.float32,
                       precision=lax.Precision.HIGHEST)


def _stage_mid(p, degi, dego, b, w):
    d = p.shape[-1]
    h = w.shape[-1]
    return pl.pallas_call(
        _stage_mid_body,
        grid=(NR // RB,),
        in_specs=[
            pl.BlockSpec((RB, d), lambda i: (i, 0)),
            pl.BlockSpec((NSC, RB, 16), lambda i: (0, i, 0)),
            pl.BlockSpec((NSC, RB, 16), lambda i: (0, i, 0)),
            pl.BlockSpec((1, d), lambda i: (0, 0)),
            pl.BlockSpec((d, h), lambda i: (0, 0)),
        ],
        out_specs=pl.BlockSpec((RB, h), lambda i: (i, 0)),
        out_shape=jax.ShapeDtypeStruct((NR, h), jnp.float32),
    )(p, degi, dego, b, w)


def _stage3_body(p_ref, degi_ref, b_ref, o_ref):
    di = degi_ref[0, :, 0:1] + degi_ref[1, :, 0:1]
    t = p_ref[...] * _norm(di) + b_ref[...]
    t = t - jnp.max(t, axis=1, keepdims=True)
    e = jnp.exp(t)
    o_ref[...] = e / jnp.sum(e, axis=1, keepdims=True)


def _stage3(p, degi, b):
    return pl.pallas_call(
        _stage3_body,
        grid=(N // RB3,),
        in_specs=[
            pl.BlockSpec((RB3, D_OUT), lambda i: (i, 0)),
            pl.BlockSpec((NSC, RB3, 16), lambda i: (0, i, 0)),
            pl.BlockSpec((1, D_OUT), lambda i: (0, 0)),
        ],
        out_specs=pl.BlockSpec((RB3, D_OUT), lambda i: (i, 0)),
        out_shape=jax.ShapeDtypeStruct((N, D_OUT), jnp.float32),
    )(p, degi, b)


def kernel(x, edge_index, W1, b1, W2, b2, W3, b3):
    src = edge_index[0]
    dst = edge_index[1]
    pad = jnp.full((CROWS * K - E,), DUMMY, jnp.int32)
    srcp2 = jnp.concatenate([src, pad]).reshape(CROWS, K)
    dstp2 = jnp.concatenate([dst, pad]).reshape(CROWS, K)
    z16 = jnp.zeros((RPT, 16), jnp.float32)
    z128 = jnp.zeros((RPT, 128), jnp.float32)
    z64 = jnp.zeros((RPT, 64), jnp.float32)

    dego, degi = _deg_kernel(srcp2, dstp2, z16)

    y1 = _stage0(x, dego, W1)
    p1 = _agg128(srcp2, dstp2, z128, y1)
    y2 = _stage_mid(p1, degi, dego, b1.reshape(1, -1), W2)
    p2 = _agg128(srcp2, dstp2, z128, y2)
    y3 = _stage_mid(p2, degi, dego, b2.reshape(1, -1), W3)
    p3 = _agg64(srcp2, dstp2, z64, y3)
    return _stage3(p3, degi, b3.reshape(1, -1))


# speedup vs baseline: 1.4859x; 1.4859x over previous
import functools

import jax
import jax.numpy as jnp
from jax import lax
from jax.experimental import pallas as pl
from jax.experimental.pallas import tpu as pltpu
from jax.experimental.pallas import tpu_sc as plsc

N = 10000
E = 320000
D_IN = 128
H1 = 128
H2 = 128
D_OUT = 64

NSC = 2
NTILE = 16
RPT = 632
NR = NTILE * RPT
DUMMY = N
K = 112
CROWS = 2880
CPT = CROWS // (NSC * NTILE)
CPA = CROWS // NTILE
EPAD = CROWS * K

RB = 632
RB3 = 1000

_MESH = plsc.VectorSubcoreMesh(core_axis_name="c", subcore_axis_name="s")


@functools.partial(
    pl.kernel,
    out_type=(
        jax.ShapeDtypeStruct((NSC, NR, 16), jnp.float32),
        jax.ShapeDtypeStruct((NSC, NR, 16), jnp.float32),
    ),
    mesh=_MESH,
    scratch_types=[
        pltpu.VMEM((CPT, K), jnp.int32),
        pltpu.VMEM((CPT, K), jnp.int32),
        pltpu.VMEM((K, 16), jnp.float32),
        pltpu.VMEM_SHARED((NR, 16), jnp.float32),
        pltpu.VMEM_SHARED((NR, 16), jnp.float32),
    ],
    compiler_params=pltpu.CompilerParams(use_tc_tiling_on_sc=False),
)
def _deg_kernel(srcp2, dstp2, z16, dego, degi, sidx, didx, ones, acc_o, acc_i):
    c = lax.axis_index("c")
    s = lax.axis_index("s")
    tid = c * NTILE + s

    pltpu.sync_copy(srcp2.at[pl.ds(tid * CPT, CPT)], sidx)
    pltpu.sync_copy(dstp2.at[pl.ds(tid * CPT, CPT)], didx)
    pltpu.sync_copy(z16, acc_o.at[pl.ds(s * RPT, RPT)])
    pltpu.sync_copy(z16, acc_i.at[pl.ds(s * RPT, RPT)])

    for r in range(K):
        ones[r] = jnp.ones((16,), jnp.float32)

    plsc.subcore_barrier()

    @pl.loop(0, CPT)
    def _(i):
        pltpu.sync_copy(ones, acc_o.at[sidx.at[i]], add=True)
        pltpu.sync_copy(ones, acc_i.at[didx.at[i]], add=True)

    plsc.subcore_barrier()
    off = s * RPT
    pltpu.sync_copy(acc_o.at[pl.ds(off, RPT)], dego.at[c, pl.ds(off, RPT)])
    pltpu.sync_copy(acc_i.at[pl.ds(off, RPT)], degi.at[c, pl.ds(off, RPT)])


def _make_agg(D):
    @functools.partial(
        pl.kernel,
        out_type=jax.ShapeDtypeStruct((NSC, NR, D), jnp.float32),
        mesh=_MESH,
        scratch_types=[
            pltpu.VMEM((CPA, K), jnp.int32),
            pltpu.VMEM((K,), jnp.int32),
            pltpu.VMEM((K,), jnp.int32),
            pltpu.VMEM((K, D), jnp.float32),
            pltpu.VMEM((K, D), jnp.float32),
            pltpu.VMEM_SHARED((NR, D), jnp.float32),
            pltpu.SemaphoreType.DMA,
            pltpu.SemaphoreType.DMA,
            pltpu.SemaphoreType.DMA,
            pltpu.SemaphoreType.DMA,
        ],
        compiler_params=pltpu.CompilerParams(use_tc_tiling_on_sc=False),
    )
    def agg(srcp2, dstp2, zrows, y, out, sidx, d0b, d1b, r0, r1, acc,
            g0, g1, dm0, dm1):
        rows = [r0, r1]
        gsem = [g0, g1]
        dbuf = [d0b, d1b]
        dsem = [dm0, dm1]
        c = lax.axis_index("c")
        s = lax.axis_index("s")

        base = s * CPA

        pltpu.sync_copy(srcp2.at[pl.ds(base, CPA)], sidx)
        pltpu.sync_copy(zrows, acc.at[pl.ds(s * RPT, RPT)])

        plsc.subcore_barrier()

        @pl.when(c == 0)
        def _():
            pltpu.async_copy(dstp2.at[base], dbuf[0], dsem[0])
            pltpu.async_copy(y.at[sidx.at[0]], rows[0], gsem[0])
            pltpu.async_copy(y.at[sidx.at[1]], rows[1], gsem[1])

            @pl.loop(0, CPA // 2)
            def _(j):
                i0 = j * 2
                for b in range(2):
                    i = i0 + b

                    @pl.when(i + 1 < CPA)
                    def _():
                        pltpu.async_copy(dstp2.at[base + i + 1], dbuf[1 - b],
                                         dsem[1 - b])

                    pltpu.make_async_copy(y.at[sidx.at[i]], rows[b],
                                          gsem[b]).wait()
                    pltpu.make_async_copy(dstp2.at[base + i], dbuf[b],
                                          dsem[b]).wait()
                    pltpu.sync_copy(rows[b], acc.at[dbuf[b]], add=True)

                    @pl.when(i + 2 < CPA)
                    def _():
                        pltpu.async_copy(y.at[sidx.at[i + 2]], rows[b],
                                         gsem[b])

        plsc.subcore_barrier()
        pltpu.sync_copy(acc.at[pl.ds(s * RPT, RPT)],
                        out.at[c, pl.ds(s * RPT, RPT)])

    return agg


_agg128 = _make_agg(128)
_agg64 = _make_agg(64)


def _norm(deg):
    return jnp.where(deg > 0, lax.rsqrt(jnp.maximum(deg, 1.0)), 0.0)


def _stage0_body(x_ref, dego_ref, w_ref, o_ref):
    deg = dego_ref[0, :, 0:1] + dego_ref[1, :, 0:1]
    h = x_ref[...] * _norm(deg)
    o_ref[...] = jnp.dot(h, w_ref[...], preferred_element_type=jnp.float32,
                         precision=lax.Precision.HIGHEST)


def _stage0(x, dego, w):
    return pl.pallas_call(
        _stage0_body,
        grid=(NR // RB,),
        in_specs=[
            pl.BlockSpec((RB, D_IN), lambda i: (i, 0)),
            pl.BlockSpec((NSC, RB, 16), lambda i: (0, i, 0)),
            pl.BlockSpec((D_IN, H1), lambda i: (0, 0)),
        ],
        out_specs=pl.BlockSpec((RB, H1), lambda i: (i, 0)),
        out_shape=jax.ShapeDtypeStruct((NR, H1), jnp.float32),
    )(x, dego, w)


def _stage_mid_body(p_ref, degi_ref, dego_ref, b_ref, w_ref, o_ref):
    di = degi_ref[0, :, 0:1] + degi_ref[1, :, 0:1]
    do = dego_ref[0, :, 0:1] + dego_ref[1, :, 0:1]
    agg = (p_ref[0] + p_ref[1]) * _norm(di) + b_ref[...]
    h = jnp.where(agg >= 0, agg, 0.1 * agg) * _norm(do)
    o_ref[...] = jnp.dot(h, w_ref[...], preferred_element_type=jnp.float32,
                         precision=lax.Precision.HIGHEST)


def _stage_mid(p, degi, dego, b, w):
    d = p.shape[-1]
    h = w.shape[-1]
    return pl.pallas_call(
        _stage_mid_body,
        grid=(NR // RB,),
        in_specs=[
            pl.BlockSpec((NSC, RB, d), lambda i: (0, i, 0)),
            pl.BlockSpec((NSC, RB, 16), lambda i: (0, i, 0)),
            pl.BlockSpec((NSC, RB, 16), lambda i: (0, i, 0)),
            pl.BlockSpec((1, d), lambda i: (0, 0)),
            pl.BlockSpec((d, h), lambda i: (0, 0)),
        ],
        out_specs=pl.BlockSpec((RB, h), lambda i: (i, 0)),
        out_shape=jax.ShapeDtypeStruct((NR, h), jnp.float32),
    )(p, degi, dego, b, w)


def _stage3_body(p_ref, degi_ref, b_ref, o_ref):
    di = degi_ref[0, :, 0:1] + degi_ref[1, :, 0:1]
    t = (p_ref[0] + p_ref[1]) * _norm(di) + b_ref[...]
    t = t - jnp.max(t, axis=1, keepdims=True)
    e = jnp.exp(t)
    o_ref[...] = e / jnp.sum(e, axis=1, keepdims=True)


def _stage3(p, degi, b):
    return pl.pallas_call(
        _stage3_body,
        grid=(N // RB3,),
        in_specs=[
            pl.BlockSpec((NSC, RB3, D_OUT), lambda i: (0, i, 0)),
            pl.BlockSpec((NSC, RB3, 16), lambda i: (0, i, 0)),
            pl.BlockSpec((1, D_OUT), lambda i: (0, 0)),
        ],
        out_specs=pl.BlockSpec((RB3, D_OUT), lambda i: (i, 0)),
        out_shape=jax.ShapeDtypeStruct((N, D_OUT), jnp.float32),
    )(p, degi, b)


def kernel(x, edge_index, W1, b1, W2, b2, W3, b3):
    src = edge_index[0]
    dst = edge_index[1]
    pad = jnp.full((CROWS * K - E,), DUMMY, jnp.int32)
    srcp2 = jnp.concatenate([src, pad]).reshape(CROWS, K)
    dstp2 = jnp.concatenate([dst, pad]).reshape(CROWS, K)
    z16 = jnp.zeros((RPT, 16), jnp.float32)
    z128 = jnp.zeros((RPT, 128), jnp.float32)
    z64 = jnp.zeros((RPT, 64), jnp.float32)

    dego, degi = _deg_kernel(srcp2, dstp2, z16)

    y1 = _stage0(x, dego, W1)
    p1 = _agg128(srcp2, dstp2, z128, y1)
    y2 = _stage_mid(p1, degi, dego, b1.reshape(1, -1), W2)
    p2 = _agg128(srcp2, dstp2, z128, y2)
    y3 = _stage_mid(p2, degi, dego, b2.reshape(1, -1), W3)
    p3 = _agg64(srcp2, dstp2, z64, y3)
    return _stage3(p3, degi, b3.reshape(1, -1))
